# Initial kernel scaffold; baseline (speedup 1.0000x reference)
import jax
import jax.numpy as jnp
from jax.experimental import pallas as pl


def kernel(inputs_raw, gate_w, W1, W2):
    raise NotImplementedError("write your pallas kernel here")


# speedup vs baseline: 1.7296x; 1.7296x over previous
import functools

import jax
import jax.numpy as jnp
from jax.experimental import pallas as pl


def _routing_kernel(x_ref, gw_ref, coef_ref):
    x = x_ref[...]
    gw = gw_ref[...]
    logits = jnp.dot(x, gw, preferred_element_type=jnp.float32)
    num_e = logits.shape[1]
    col = jax.lax.broadcasted_iota(jnp.int32, logits.shape, 1)
    m1 = jnp.max(logits, axis=1, keepdims=True)
    e1 = jnp.min(jnp.where(logits == m1, col, num_e), axis=1, keepdims=True)
    oh1 = col == e1
    logits2 = jnp.where(oh1, -jnp.inf, logits)
    m2 = jnp.max(logits2, axis=1, keepdims=True)
    e2 = jnp.min(jnp.where(logits2 == m2, col, num_e), axis=1, keepdims=True)
    oh2 = col == e2
    b = jnp.exp(m2 - m1)
    w1 = 1.0 / (1.0 + b)
    w2 = b / (1.0 + b)
    coef_ref[...] = jnp.where(oh1, w1, 0.0) + jnp.where(oh2, w2, 0.0)


def _dense_ffn_kernel(x_ref, coef_ref, w1_ref, w2_ref, out_ref):
    e = pl.program_id(0)
    x = x_ref[...]
    h = jnp.dot(x, w1_ref[0], preferred_element_type=jnp.float32)
    h = h * jax.nn.sigmoid(h)
    y = jnp.dot(h, w2_ref[0], preferred_element_type=jnp.float32)
    col = jax.lax.broadcasted_iota(jnp.int32, coef_ref.shape, 1)
    c = jnp.sum(jnp.where(col == e, coef_ref[...], 0.0), axis=1, keepdims=True)

    @pl.when(e == 0)
    def _():
        out_ref[...] = jnp.zeros_like(out_ref)

    out_ref[...] += c * y


def kernel(inputs_raw, gate_w, W1, W2):
    ishape = inputs_raw.shape
    x = inputs_raw.reshape(-1, ishape[-1])
    n_tok, d_model = x.shape
    num_e = gate_w.shape[1]
    d_ff = W1.shape[2]

    coef = pl.pallas_call(
        _routing_kernel,
        out_shape=jax.ShapeDtypeStruct((n_tok, num_e), jnp.float32),
    )(x, gate_w)

    out = pl.pallas_call(
        _dense_ffn_kernel,
        grid=(num_e,),
        in_specs=[
            pl.BlockSpec((n_tok, d_model), lambda e: (0, 0)),
            pl.BlockSpec((n_tok, num_e), lambda e: (0, 0)),
            pl.BlockSpec((1, d_model, d_ff), lambda e: (e, 0, 0)),
            pl.BlockSpec((1, d_ff, d_model), lambda e: (e, 0, 0)),
        ],
        out_specs=pl.BlockSpec((n_tok, d_model), lambda e: (0, 0)),
        out_shape=jax.ShapeDtypeStruct((n_tok, d_model), jnp.float32),
    )(x, coef, W1, W2)

    return out.reshape(ishape)


# speedup vs baseline: 5.1457x; 2.9751x over previous
import functools

import jax
import jax.numpy as jnp
from jax import lax
from jax.experimental import pallas as pl
from jax.experimental.pallas import tpu as pltpu
from jax.experimental.pallas import tpu_sc as plsc

_RB = 128
_SEG_ALIGN = 8
_NW = 32
_LANES = 16


def _routing_kernel(x_ref, gw_ref, post_ref, wt_ref, counts_ref, offsets_ref):
    x = x_ref[...]
    gw = gw_ref[...]
    n_tok = x.shape[0]
    logits = jnp.dot(x, gw, preferred_element_type=jnp.float32)
    num_e = logits.shape[1]
    col = lax.broadcasted_iota(jnp.int32, logits.shape, 1)
    m1 = jnp.max(logits, axis=1, keepdims=True)
    e1 = jnp.min(jnp.where(logits == m1, col, num_e), axis=1, keepdims=True)
    oh1 = col == e1
    logits2 = jnp.where(oh1, -jnp.inf, logits)
    m2 = jnp.max(logits2, axis=1, keepdims=True)
    e2 = jnp.min(jnp.where(logits2 == m2, col, num_e), axis=1, keepdims=True)
    oh2 = col == e2
    b = jnp.exp(m2 - m1)
    w1 = 1.0 / (1.0 + b)
    w2 = b / (1.0 + b)

    oh = jnp.where(oh1 | oh2, 1.0, 0.0)
    n_chunk = 8
    cs = n_tok // n_chunk
    r = lax.broadcasted_iota(jnp.int32, (cs, cs), 0)
    c = lax.broadcasted_iota(jnp.int32, (cs, cs), 1)
    lower = jnp.where(r > c, 1.0, 0.0)
    tot = jnp.zeros((1, num_e), jnp.float32)
    chunks = []
    for k in range(n_chunk):
        blk = oh[k * cs:(k + 1) * cs, :]
        chunks.append(
            jnp.dot(lower, blk, preferred_element_type=jnp.float32) + tot)
        tot = tot + jnp.sum(blk, axis=0, keepdims=True)
    cum = jnp.concatenate(chunks, axis=0)

    align = float(_SEG_ALIGN)
    cnt_pad = jnp.ceil(tot / align) * align
    re = lax.broadcasted_iota(jnp.int32, (num_e, num_e), 0)
    ce = lax.broadcasted_iota(jnp.int32, (num_e, num_e), 1)
    lower_e = jnp.where(re < ce, 1.0, 0.0)
    off_row = jnp.dot(cnt_pad, lower_e, preferred_element_type=jnp.float32)

    dest = cum + off_row
    pos0 = jnp.sum(jnp.where(oh1, dest, 0.0), axis=1)
    pos1 = jnp.sum(jnp.where(oh2, dest, 0.0), axis=1)
    post_ref[0, :] = pos0.astype(jnp.int32)
    post_ref[1, :] = pos1.astype(jnp.int32)
    wt_ref[0, :] = w1[:, 0]
    wt_ref[1, :] = w2[:, 0]
    counts_ref[...] = tot.astype(jnp.int32)
    offsets_ref[...] = off_row.astype(jnp.int32)


def _dispatch_kernel(x_hbm, pos0_hbm, pos1_hbm, xg_hbm,
                     rows_v, pos0_v, pos1_v, sem):
    wid = lax.axis_index("s") * 2 + lax.axis_index("c")
    tpw = pos0_v.shape[0]
    base = wid * tpw
    pltpu.sync_copy(pos0_hbm.at[pl.ds(base, tpw)], pos0_v)
    pltpu.sync_copy(pos1_hbm.at[pl.ds(base, tpw)], pos1_v)
    pltpu.sync_copy(x_hbm.at[pl.ds(base, tpw)], rows_v)
    cp0 = pltpu.async_copy(rows_v, xg_hbm.at[pos0_v], sem)
    cp1 = pltpu.async_copy(rows_v, xg_hbm.at[pos1_v], sem)
    cp0.wait()
    cp1.wait()


def _ragged_ffn_kernel(counts_ref, offsets_ref, xg_ref, w1_ref, w2_ref,
                       yg_ref):
    e = pl.program_id(0)
    cnt = counts_ref[0, e]
    off = offsets_ref[0, e]
    nb = (cnt + _RB - 1) // _RB

    def body(j, _):
        s = pl.multiple_of(off + j * _RB, _SEG_ALIGN)
        xs = xg_ref[pl.ds(s, _RB), :]
        h = jnp.dot(xs, w1_ref[0], preferred_element_type=jnp.float32)
        h = h * jax.nn.sigmoid(h)
        ys = jnp.dot(h, w2_ref[0], preferred_element_type=jnp.float32)
        yg_ref[pl.ds(s, _RB), :] = ys
        return 0

    lax.fori_loop(0, nb, body, 0)


def _combine_kernel(yg_hbm, pos0_hbm, pos1_hbm, w0_hbm, w1_hbm, out_hbm,
                    pos0_v, pos1_v, w0_v, w1_v, rows0_v, rows1_v, sem):
    wid = lax.axis_index("s") * 2 + lax.axis_index("c")
    tpw = pos0_v.shape[0]
    d_model = rows0_v.shape[1]
    base = wid * tpw
    pltpu.sync_copy(pos0_hbm.at[pl.ds(base, tpw)], pos0_v)
    pltpu.sync_copy(pos1_hbm.at[pl.ds(base, tpw)], pos1_v)
    pltpu.sync_copy(w0_hbm.at[pl.ds(base, tpw)], w0_v)
    pltpu.sync_copy(w1_hbm.at[pl.ds(base, tpw)], w1_v)
    g0 = pltpu.async_copy(yg_hbm.at[pos0_v], rows0_v, sem)
    g1 = pltpu.async_copy(yg_hbm.at[pos1_v], rows1_v, sem)
    g0.wait()
    g1.wait()

    lane = lax.broadcasted_iota(jnp.int32, (_LANES,), 0)

    def tok_body(t, _):
        t16 = jnp.full((_LANES,), t, jnp.int32)
        w0b = plsc.load_gather(w0_v, [t16])
        w1b = plsc.load_gather(w1_v, [t16])
        for k in range(d_model // _LANES):
            idx = k * _LANES + lane
            r0 = plsc.load_gather(rows0_v, [t16, idx])
            r1 = plsc.load_gather(rows1_v, [t16, idx])
            plsc.store_scatter(rows0_v, [t16, idx], w0b * r0 + w1b * r1)
        return 0

    lax.fori_loop(0, tpw, tok_body, 0)
    pltpu.sync_copy(rows0_v, out_hbm.at[pl.ds(base, tpw)])


def kernel(inputs_raw, gate_w, W1, W2):
    ishape = inputs_raw.shape
    x = inputs_raw.reshape(-1, ishape[-1])
    n_tok, d_model = x.shape
    num_e = gate_w.shape[1]
    d_ff = W1.shape[2]
    n_pad = 2 * n_tok + num_e * _SEG_ALIGN + _RB
    tpw = n_tok // _NW

    post, wt, counts, offsets = pl.pallas_call(
        _routing_kernel,
        out_shape=(
            jax.ShapeDtypeStruct((2, n_tok), jnp.int32),
            jax.ShapeDtypeStruct((2, n_tok), jnp.float32),
            jax.ShapeDtypeStruct((1, num_e), jnp.int32),
            jax.ShapeDtypeStruct((1, num_e), jnp.int32),
        ),
    )(x, gate_w)

    pos0, pos1 = post[0], post[1]
    w0, w1 = wt[0], wt[1]

    mesh = plsc.VectorSubcoreMesh(
        core_axis_name="c", subcore_axis_name="s", num_cores=2,
        num_subcores=16)

    xg = pl.kernel(
        _dispatch_kernel,
        out_type=jax.ShapeDtypeStruct((n_pad, d_model), jnp.float32),
        mesh=mesh,
        scratch_types=[
            pltpu.VMEM((tpw, d_model), jnp.float32),
            pltpu.VMEM((tpw,), jnp.int32),
            pltpu.VMEM((tpw,), jnp.int32),
            pltpu.SemaphoreType.DMA,
        ],
    )(x, pos0, pos1)

    yg = pl.pallas_call(
        _ragged_ffn_kernel,
        grid=(num_e,),
        in_specs=[
            pl.BlockSpec(memory_space=pltpu.SMEM),
            pl.BlockSpec(memory_space=pltpu.SMEM),
            pl.BlockSpec((n_pad, d_model), lambda e: (0, 0)),
            pl.BlockSpec((1, d_model, d_ff), lambda e: (e, 0, 0)),
            pl.BlockSpec((1, d_ff, d_model), lambda e: (e, 0, 0)),
        ],
        out_specs=pl.BlockSpec((n_pad, d_model), lambda e: (0, 0)),
        out_shape=jax.ShapeDtypeStruct((n_pad, d_model), jnp.float32),
    )(counts, offsets, xg, W1, W2)

    out = pl.kernel(
        _combine_kernel,
        out_type=jax.ShapeDtypeStruct((n_tok, d_model), jnp.float32),
        mesh=mesh,
        compiler_params=pltpu.CompilerParams(needs_layout_passes=False),
        scratch_types=[
            pltpu.VMEM((tpw,), jnp.int32),
            pltpu.VMEM((tpw,), jnp.int32),
            pltpu.VMEM((tpw,), jnp.float32),
            pltpu.VMEM((tpw,), jnp.float32),
            pltpu.VMEM((tpw, d_model), jnp.float32),
            pltpu.VMEM((tpw, d_model), jnp.float32),
            pltpu.SemaphoreType.DMA,
        ],
    )(yg, pos0, pos1, w0, w1)

    return out.reshape(ishape)
